# baseline (speedup 1.0000x reference)
import functools

import jax
import jax.numpy as jnp
from jax import lax
from jax.experimental import pallas as pl
from jax.experimental.pallas import tpu as pltpu
from jax.experimental.pallas import tpu_sc as plsc

N = 10000
D = 128
E = 320000
NC = 2
NS = 16
EPT = E // (NC * NS)
K = 128
SCH = 80
EPTS = SCH * K
DUMP_ROW = 10016
NROWS = 10240
ROWS_PER_TILE = NROWS // NS
NPAD = 10240
DEG_STRIPE = NPAD // NS

_mesh = plsc.VectorSubcoreMesh(core_axis_name="c", subcore_axis_name="s")


@functools.partial(
    pl.kernel,
    out_type=jax.ShapeDtypeStruct((NC, NPAD), jnp.float32),
    mesh=_mesh,
    compiler_params=pltpu.CompilerParams(needs_layout_passes=False),
    scratch_types=[
        pltpu.VMEM((NPAD,), jnp.float32),
        pltpu.VMEM((EPT,), jnp.int32),
        pltpu.VMEM((NS, DEG_STRIPE), jnp.float32),
        pltpu.VMEM((DEG_STRIPE,), jnp.float32),
        pltpu.VMEM_SHARED((NS, NPAD), jnp.float32),
    ],
)
def _deg_kernel(dst_hbm, out_hbm, pdeg, didx, sbuf, obuf, shared_deg):
    cid = lax.axis_index("c")
    sid = lax.axis_index("s")
    tid = cid * NS + sid

    zero16 = jnp.zeros((16,), jnp.float32)

    @pl.loop(0, NPAD // 16)
    def _(r):
        pdeg[pl.ds(r * 16, 16)] = zero16

    pltpu.sync_copy(dst_hbm.at[pl.ds(tid * EPT, EPT)], didx)
    ones16 = jnp.ones((16,), jnp.float32)

    @pl.loop(0, EPT // 16)
    def _(j):
        idx16 = didx[pl.ds(j * 16, 16)]
        plsc.addupdate_scatter(pdeg, [idx16], ones16)

    pltpu.sync_copy(pdeg, shared_deg.at[sid])
    plsc.subcore_barrier()
    pltpu.sync_copy(shared_deg.at[:, pl.ds(sid * DEG_STRIPE, DEG_STRIPE)], sbuf)

    @pl.loop(0, DEG_STRIPE // 16)
    def _(c):
        acc16 = sbuf[0, pl.ds(c * 16, 16)]
        for w in range(1, NS):
            acc16 += sbuf[w, pl.ds(c * 16, 16)]
        obuf[pl.ds(c * 16, 16)] = acc16

    pltpu.sync_copy(obuf, out_hbm.at[cid, pl.ds(sid * DEG_STRIPE, DEG_STRIPE)])


@functools.partial(
    pl.kernel,
    out_type=jax.ShapeDtypeStruct((NC, NROWS, D), jnp.float32),
    mesh=_mesh,
    compiler_params=pltpu.CompilerParams(needs_layout_passes=False),
    scratch_types=[
        pltpu.VMEM((K,), jnp.int32),
        pltpu.VMEM((K,), jnp.int32),
        pltpu.VMEM((K,), jnp.int32),
        pltpu.VMEM((K,), jnp.int32),
        pltpu.VMEM((K, D), jnp.float32),
        pltpu.VMEM((K, D), jnp.float32),
        pltpu.VMEM_SHARED((NROWS, D), jnp.float32),
        pltpu.SemaphoreType.DMA,
        pltpu.SemaphoreType.DMA,
        pltpu.SemaphoreType.DMA,
        pltpu.SemaphoreType.DMA,
        pltpu.SemaphoreType.DMA,
        pltpu.SemaphoreType.DMA,
    ],
)
def _prop_kernel(g_hbm, src_hbm, dst_hbm, zeros_hbm, out_hbm,
                 sidx0, sidx1, didx0, didx1, rows0, rows1, acc,
                 isem0, isem1, dsem0, dsem1, gsem0, gsem1):
    cid = lax.axis_index("c")
    sid = lax.axis_index("s")
    tid = cid * NS + sid
    base = pl.multiple_of(tid * EPTS, 8)

    pltpu.sync_copy(src_hbm.at[pl.ds(base, K)], sidx0)
    pltpu.async_copy(dst_hbm.at[pl.ds(base, K)], didx0, dsem0)
    pltpu.async_copy(src_hbm.at[pl.ds(base + K, K)], sidx1, isem1)
    pltpu.async_copy(dst_hbm.at[pl.ds(base + K, K)], didx1, dsem1)
    pltpu.sync_copy(zeros_hbm, acc.at[pl.ds(sid * ROWS_PER_TILE, ROWS_PER_TILE)])
    pltpu.async_copy(g_hbm.at[sidx0], rows0, gsem0)
    plsc.subcore_barrier()

    @pl.loop(0, SCH // 2 - 1)
    def _(i):
        c = i * 2
        o2 = pl.multiple_of((c + 2) * K, 8)
        o3 = pl.multiple_of((c + 3) * K, 8)
        pltpu.make_async_copy(src_hbm.at[pl.ds(base, K)], sidx1, isem1).wait()
        pltpu.async_copy(g_hbm.at[sidx1], rows1, gsem1)
        pltpu.make_async_copy(g_hbm.at[sidx0], rows0, gsem0).wait()
        pltpu.async_copy(src_hbm.at[pl.ds(base + o2, K)], sidx0, isem0)
        pltpu.make_async_copy(dst_hbm.at[pl.ds(base, K)], didx0, dsem0).wait()
        pltpu.sync_copy(rows0, acc.at[didx0], add=True)
        pltpu.async_copy(dst_hbm.at[pl.ds(base + o2, K)], didx0, dsem0)
        pltpu.make_async_copy(src_hbm.at[pl.ds(base, K)], sidx0, isem0).wait()
        pltpu.async_copy(g_hbm.at[sidx0], rows0, gsem0)
        pltpu.make_async_copy(g_hbm.at[sidx1], rows1, gsem1).wait()
        pltpu.async_copy(src_hbm.at[pl.ds(base + o3, K)], sidx1, isem1)
        pltpu.make_async_copy(dst_hbm.at[pl.ds(base, K)], didx1, dsem1).wait()
        pltpu.sync_copy(rows1, acc.at[didx1], add=True)
        pltpu.async_copy(dst_hbm.at[pl.ds(base + o3, K)], didx1, dsem1)

    pltpu.make_async_copy(src_hbm.at[pl.ds(base, K)], sidx1, isem1).wait()
    pltpu.async_copy(g_hbm.at[sidx1], rows1, gsem1)
    pltpu.make_async_copy(g_hbm.at[sidx0], rows0, gsem0).wait()
    pltpu.make_async_copy(dst_hbm.at[pl.ds(base, K)], didx0, dsem0).wait()
    pltpu.sync_copy(rows0, acc.at[didx0], add=True)
    pltpu.make_async_copy(g_hbm.at[sidx1], rows1, gsem1).wait()
    pltpu.make_async_copy(dst_hbm.at[pl.ds(base, K)], didx1, dsem1).wait()
    pltpu.sync_copy(rows1, acc.at[didx1], add=True)

    plsc.subcore_barrier()
    pltpu.sync_copy(acc.at[pl.ds(sid * ROWS_PER_TILE, ROWS_PER_TILE)],
                    out_hbm.at[cid, pl.ds(sid * ROWS_PER_TILE, ROWS_PER_TILE)])


BS = 1000
GR = N // BS


def _dinv_body(degp_ref, o_ref):
    deg = degp_ref[0] + degp_ref[1] + 1.0
    o_ref[...] = lax.rsqrt(deg)


def _dinv(degp):
    return pl.pallas_call(
        _dinv_body,
        out_shape=jax.ShapeDtypeStruct((NPAD // 128, 128), jnp.float32),
    )(degp.reshape(NC, NPAD // 128, 128))


def _mm_scale_body(x_ref, w_ref, dinv_ref, o_ref):
    h = jnp.dot(x_ref[...], w_ref[...], preferred_element_type=jnp.float32)
    o_ref[...] = dinv_ref[...] * h


def _mm_scale(x, w, dinv):
    return pl.pallas_call(
        _mm_scale_body,
        grid=(GR,),
        in_specs=[
            pl.BlockSpec((BS, D), lambda i: (i, 0)),
            pl.BlockSpec((D, D), lambda i: (0, 0)),
            pl.BlockSpec((BS, 1), lambda i: (i, 0)),
        ],
        out_specs=pl.BlockSpec((BS, D), lambda i: (i, 0)),
        out_shape=jax.ShapeDtypeStruct((N, D), jnp.float32),
    )(x, w, dinv)


def _layer2_body(p0_ref, p1_ref, g_ref, dinv_ref, b_ref, w_ref, o_ref):
    z = dinv_ref[...] * (p0_ref[0] + p1_ref[0] + g_ref[...]) + b_ref[...]
    z = jnp.maximum(z, 0.0)
    h = jnp.dot(z, w_ref[...], preferred_element_type=jnp.float32)
    o_ref[...] = dinv_ref[...] * h


def _layer2(p, g, dinv, b, w):
    return pl.pallas_call(
        _layer2_body,
        grid=(GR,),
        in_specs=[
            pl.BlockSpec((1, BS, D), lambda i: (0, i, 0)),
            pl.BlockSpec((1, BS, D), lambda i: (1, i, 0)),
            pl.BlockSpec((BS, D), lambda i: (i, 0)),
            pl.BlockSpec((BS, 1), lambda i: (i, 0)),
            pl.BlockSpec((1, D), lambda i: (0, 0)),
            pl.BlockSpec((D, D), lambda i: (0, 0)),
        ],
        out_specs=pl.BlockSpec((BS, D), lambda i: (i, 0)),
        out_shape=jax.ShapeDtypeStruct((N, D), jnp.float32),
    )(p, p, g, dinv, b, w)


def _final_body(p0_ref, p1_ref, g_ref, dinv_ref, b_ref, o_ref):
    z = dinv_ref[...] * (p0_ref[0] + p1_ref[0] + g_ref[...]) + b_ref[...]
    m = jnp.max(z, axis=1, keepdims=True)
    zc = z - m
    o_ref[...] = zc - jnp.log(jnp.sum(jnp.exp(zc), axis=1, keepdims=True))


def _final(p, g, dinv, b):
    return pl.pallas_call(
        _final_body,
        grid=(GR,),
        in_specs=[
            pl.BlockSpec((1, BS, D), lambda i: (0, i, 0)),
            pl.BlockSpec((1, BS, D), lambda i: (1, i, 0)),
            pl.BlockSpec((BS, D), lambda i: (i, 0)),
            pl.BlockSpec((BS, 1), lambda i: (i, 0)),
            pl.BlockSpec((1, D), lambda i: (0, 0)),
        ],
        out_specs=pl.BlockSpec((BS, D), lambda i: (i, 0)),
        out_shape=jax.ShapeDtypeStruct((N, D), jnp.float32),
    )(p, p, g, dinv, b)


def kernel(x, adj, W1, b1, W2, b2):
    src = adj[0].astype(jnp.int32)
    dst = adj[1].astype(jnp.int32)
    srcp = jnp.concatenate(
        [src.reshape(NC * NS, EPT),
         jnp.zeros((NC * NS, EPTS - EPT), jnp.int32)], axis=1).reshape(-1)
    dstp = jnp.concatenate(
        [dst.reshape(NC * NS, EPT),
         jnp.full((NC * NS, EPTS - EPT), DUMP_ROW, jnp.int32)], axis=1).reshape(-1)
    zeros = jnp.zeros((ROWS_PER_TILE, D), jnp.float32)

    degp = _deg_kernel(dst)
    dinv = _dinv(degp).reshape(-1)[:N].reshape(N, 1)

    b1r = b1.reshape(1, D)
    b2r = b2.reshape(1, D)

    g1 = _mm_scale(x, W1, dinv)
    p = _prop_kernel(g1, srcp, dstp, zeros)
    g2 = _layer2(p, g1, dinv, b1r, W2)
    q = _prop_kernel(g2, srcp, dstp, zeros)
    return _final(q, g2, dinv, b2r)


# speedup vs baseline: 2.7450x; 2.7450x over previous
import functools

import jax
import jax.numpy as jnp
from jax import lax
from jax.experimental import pallas as pl
from jax.experimental.pallas import tpu as pltpu
from jax.experimental.pallas import tpu_sc as plsc

N = 10000
D = 128
E = 320000
NC = 2
NS = 16
EPT = E // (NC * NS)
K = 80
NCHUNK = EPT // K
NROWS = 10240
ROWS_PER_TILE = NROWS // NS
NPAD = 10240
DEG_STRIPE = NPAD // NS

_mesh = plsc.VectorSubcoreMesh(core_axis_name="c", subcore_axis_name="s")


@functools.partial(
    pl.kernel,
    out_type=jax.ShapeDtypeStruct((NC, NPAD), jnp.float32),
    mesh=_mesh,
    compiler_params=pltpu.CompilerParams(needs_layout_passes=False),
    scratch_types=[
        pltpu.VMEM((NPAD,), jnp.float32),
        pltpu.VMEM((EPT,), jnp.int32),
        pltpu.VMEM((NS, DEG_STRIPE), jnp.float32),
        pltpu.VMEM((DEG_STRIPE,), jnp.float32),
        pltpu.VMEM_SHARED((NS, NPAD), jnp.float32),
    ],
)
def _deg_kernel(dst_hbm, out_hbm, pdeg, didx, sbuf, obuf, shared_deg):
    cid = lax.axis_index("c")
    sid = lax.axis_index("s")
    tid = cid * NS + sid

    zero16 = jnp.zeros((16,), jnp.float32)

    @pl.loop(0, NPAD // 16)
    def _(r):
        pdeg[pl.ds(r * 16, 16)] = zero16

    pltpu.sync_copy(dst_hbm.at[pl.ds(tid * EPT, EPT)], didx)
    ones16 = jnp.ones((16,), jnp.float32)

    @pl.loop(0, EPT // 16)
    def _(j):
        idx16 = didx[pl.ds(j * 16, 16)]
        plsc.addupdate_scatter(pdeg, [idx16], ones16)

    pltpu.sync_copy(pdeg, shared_deg.at[sid])
    plsc.subcore_barrier()
    pltpu.sync_copy(shared_deg.at[:, pl.ds(sid * DEG_STRIPE, DEG_STRIPE)], sbuf)

    @pl.loop(0, DEG_STRIPE // 16)
    def _(c):
        acc16 = sbuf[0, pl.ds(c * 16, 16)]
        for w in range(1, NS):
            acc16 += sbuf[w, pl.ds(c * 16, 16)]
        obuf[pl.ds(c * 16, 16)] = acc16

    pltpu.sync_copy(obuf, out_hbm.at[cid, pl.ds(sid * DEG_STRIPE, DEG_STRIPE)])


@functools.partial(
    pl.kernel,
    out_type=jax.ShapeDtypeStruct((NC, NROWS, D), jnp.float32),
    mesh=_mesh,
    compiler_params=pltpu.CompilerParams(needs_layout_passes=False),
    scratch_types=[
        pltpu.VMEM((EPT,), jnp.int32),
        pltpu.VMEM((K,), jnp.int32),
        pltpu.VMEM((K,), jnp.int32),
        pltpu.VMEM((K, D), jnp.float32),
        pltpu.VMEM((K, D), jnp.float32),
        pltpu.VMEM_SHARED((NROWS, D), jnp.float32),
        pltpu.SemaphoreType.DMA,
        pltpu.SemaphoreType.DMA,
        pltpu.SemaphoreType.DMA,
        pltpu.SemaphoreType.DMA,
    ],
)
def _prop_kernel(g_hbm, src_hbm, dst_hbm, out_hbm,
                 sidx, didx0, didx1, rows0, rows1, acc,
                 gsem0, gsem1, dsem0, dsem1):
    cid = lax.axis_index("c")
    sid = lax.axis_index("s")
    tid = cid * NS + sid
    base = pl.multiple_of(tid * EPT, 8)

    zero16 = jnp.zeros((16,), jnp.float32)

    @pl.loop(0, K)
    def _(r):
        for cc in range(D // 16):
            rows0[r, pl.ds(cc * 16, 16)] = zero16

    for j in range(ROWS_PER_TILE // K):
        pltpu.sync_copy(rows0, acc.at[pl.ds(sid * ROWS_PER_TILE + j * K, K)])
    pltpu.sync_copy(src_hbm.at[pl.ds(base, EPT)], sidx)
    pltpu.async_copy(dst_hbm.at[pl.ds(base, K)], didx0, dsem0)
    pltpu.async_copy(g_hbm.at[sidx.at[pl.ds(0, K)]], rows0, gsem0)
    plsc.subcore_barrier()

    @pl.loop(0, NCHUNK // 2)
    def _(i):
        c = i * 2
        o1 = pl.multiple_of((c + 1) * K, 8)
        o2 = pl.multiple_of((c + 2) * K, 8)
        pltpu.async_copy(dst_hbm.at[pl.ds(base + o1, K)], didx1, dsem1)
        pltpu.async_copy(g_hbm.at[sidx.at[pl.ds(o1, K)]], rows1, gsem1)
        pltpu.make_async_copy(g_hbm.at[sidx.at[pl.ds(0, K)]], rows0, gsem0).wait()
        pltpu.make_async_copy(dst_hbm.at[pl.ds(base, K)], didx0, dsem0).wait()
        pltpu.sync_copy(rows0, acc.at[didx0], add=True)
        pltpu.async_copy(dst_hbm.at[pl.ds(base + o2, K)], didx0, dsem0)
        pltpu.async_copy(g_hbm.at[sidx.at[pl.ds(o2, K)]], rows0, gsem0)
        pltpu.make_async_copy(g_hbm.at[sidx.at[pl.ds(0, K)]], rows1, gsem1).wait()
        pltpu.make_async_copy(dst_hbm.at[pl.ds(base, K)], didx1, dsem1).wait()
        pltpu.sync_copy(rows1, acc.at[didx1], add=True)

    pltpu.make_async_copy(g_hbm.at[sidx.at[pl.ds(0, K)]], rows0, gsem0).wait()
    pltpu.make_async_copy(dst_hbm.at[pl.ds(base, K)], didx0, dsem0).wait()
    pltpu.sync_copy(rows0, acc.at[didx0], add=True)

    plsc.subcore_barrier()
    pltpu.sync_copy(acc.at[pl.ds(sid * ROWS_PER_TILE, ROWS_PER_TILE)],
                    out_hbm.at[cid, pl.ds(sid * ROWS_PER_TILE, ROWS_PER_TILE)])


BS = 1000
GR = N // BS


def _mm_scale_body(x_ref, w_ref, d0_ref, d1_ref, o_ref):
    dinv = lax.rsqrt(d0_ref[0] + d1_ref[0] + 1.0)
    h = jnp.dot(x_ref[...], w_ref[...], preferred_element_type=jnp.float32)
    o_ref[...] = dinv * h


def _mm_scale(x, w, degp):
    return pl.pallas_call(
        _mm_scale_body,
        grid=(GR,),
        in_specs=[
            pl.BlockSpec((BS, D), lambda i: (i, 0)),
            pl.BlockSpec((D, D), lambda i: (0, 0)),
            pl.BlockSpec((1, BS, 1), lambda i: (0, i, 0)),
            pl.BlockSpec((1, BS, 1), lambda i: (1, i, 0)),
        ],
        out_specs=pl.BlockSpec((BS, D), lambda i: (i, 0)),
        out_shape=jax.ShapeDtypeStruct((N, D), jnp.float32),
    )(x, w, degp, degp)


def _layer2_body(p0_ref, p1_ref, g_ref, d0_ref, d1_ref, b_ref, w_ref, o_ref):
    dinv = lax.rsqrt(d0_ref[0] + d1_ref[0] + 1.0)
    z = dinv * (p0_ref[0] + p1_ref[0] + g_ref[...]) + b_ref[...]
    z = jnp.maximum(z, 0.0)
    h = jnp.dot(z, w_ref[...], preferred_element_type=jnp.float32)
    o_ref[...] = dinv * h


def _layer2(p, g, degp, b, w):
    return pl.pallas_call(
        _layer2_body,
        grid=(GR,),
        in_specs=[
            pl.BlockSpec((1, BS, D), lambda i: (0, i, 0)),
            pl.BlockSpec((1, BS, D), lambda i: (1, i, 0)),
            pl.BlockSpec((BS, D), lambda i: (i, 0)),
            pl.BlockSpec((1, BS, 1), lambda i: (0, i, 0)),
            pl.BlockSpec((1, BS, 1), lambda i: (1, i, 0)),
            pl.BlockSpec((1, D), lambda i: (0, 0)),
            pl.BlockSpec((D, D), lambda i: (0, 0)),
        ],
        out_specs=pl.BlockSpec((BS, D), lambda i: (i, 0)),
        out_shape=jax.ShapeDtypeStruct((N, D), jnp.float32),
    )(p, p, g, degp, degp, b, w)


def _final_body(p0_ref, p1_ref, g_ref, d0_ref, d1_ref, b_ref, o_ref):
    dinv = lax.rsqrt(d0_ref[0] + d1_ref[0] + 1.0)
    z = dinv * (p0_ref[0] + p1_ref[0] + g_ref[...]) + b_ref[...]
    m = jnp.max(z, axis=1, keepdims=True)
    zc = z - m
    o_ref[...] = zc - jnp.log(jnp.sum(jnp.exp(zc), axis=1, keepdims=True))


def _final(p, g, degp, b):
    return pl.pallas_call(
        _final_body,
        grid=(GR,),
        in_specs=[
            pl.BlockSpec((1, BS, D), lambda i: (0, i, 0)),
            pl.BlockSpec((1, BS, D), lambda i: (1, i, 0)),
            pl.BlockSpec((BS, D), lambda i: (i, 0)),
            pl.BlockSpec((1, BS, 1), lambda i: (0, i, 0)),
            pl.BlockSpec((1, BS, 1), lambda i: (1, i, 0)),
            pl.BlockSpec((1, D), lambda i: (0, 0)),
        ],
        out_specs=pl.BlockSpec((BS, D), lambda i: (i, 0)),
        out_shape=jax.ShapeDtypeStruct((N, D), jnp.float32),
    )(p, p, g, degp, degp, b)


def kernel(x, adj, W1, b1, W2, b2):
    src = adj[0].astype(jnp.int32)
    dst = adj[1].astype(jnp.int32)
    degp = _deg_kernel(dst)
    deg3 = degp.reshape(NC, NPAD, 1)

    b1r = b1.reshape(1, D)
    b2r = b2.reshape(1, D)

    g1 = _mm_scale(x, W1, deg3)
    p = _prop_kernel(g1, src, dst)
    g2 = _layer2(p, g1, deg3, b1r, W2)
    q = _prop_kernel(g2, src, dst)
    return _final(q, g2, deg3, b2r)


# speedup vs baseline: 2.8023x; 1.0209x over previous
import functools

import jax
import jax.numpy as jnp
from jax import lax
from jax.experimental import pallas as pl
from jax.experimental.pallas import tpu as pltpu
from jax.experimental.pallas import tpu_sc as plsc

N = 10000
D = 128
E = 320000
NC = 2
NS = 16
EPT = E // (NC * NS)
K = 80
NCHUNK = EPT // K
NROWS = 10240
ROWS_PER_TILE = NROWS // NS
NPAD = 10240
DEG_STRIPE = NPAD // NS

_mesh = plsc.VectorSubcoreMesh(core_axis_name="c", subcore_axis_name="s")


@functools.partial(
    pl.kernel,
    out_type=jax.ShapeDtypeStruct((NC, NPAD), jnp.float32),
    mesh=_mesh,
    compiler_params=pltpu.CompilerParams(needs_layout_passes=False),
    scratch_types=[
        pltpu.VMEM((NPAD,), jnp.float32),
        pltpu.VMEM((EPT,), jnp.int32),
        pltpu.VMEM((NS, DEG_STRIPE), jnp.float32),
        pltpu.VMEM((DEG_STRIPE,), jnp.float32),
        pltpu.VMEM_SHARED((NS, NPAD), jnp.float32),
    ],
)
def _deg_kernel(dst_hbm, out_hbm, pdeg, didx, sbuf, obuf, shared_deg):
    cid = lax.axis_index("c")
    sid = lax.axis_index("s")
    tid = cid * NS + sid

    zero16 = jnp.zeros((16,), jnp.float32)

    @pl.loop(0, NPAD // 16)
    def _(r):
        pdeg[pl.ds(r * 16, 16)] = zero16

    pltpu.sync_copy(dst_hbm.at[pl.ds(tid * EPT, EPT)], didx)
    ones16 = jnp.ones((16,), jnp.float32)

    @pl.loop(0, EPT // 16)
    def _(j):
        idx16 = didx[pl.ds(j * 16, 16)]
        plsc.addupdate_scatter(pdeg, [idx16], ones16)

    pltpu.sync_copy(pdeg, shared_deg.at[sid])
    plsc.subcore_barrier()
    pltpu.sync_copy(shared_deg.at[:, pl.ds(sid * DEG_STRIPE, DEG_STRIPE)], sbuf)

    @pl.loop(0, DEG_STRIPE // 16)
    def _(c):
        acc16 = sbuf[0, pl.ds(c * 16, 16)]
        for w in range(1, NS):
            acc16 += sbuf[w, pl.ds(c * 16, 16)]
        obuf[pl.ds(c * 16, 16)] = acc16

    pltpu.sync_copy(obuf, out_hbm.at[cid, pl.ds(sid * DEG_STRIPE, DEG_STRIPE)])


@functools.partial(
    pl.kernel,
    out_type=jax.ShapeDtypeStruct((NC, NROWS, D), jnp.float32),
    mesh=_mesh,
    compiler_params=pltpu.CompilerParams(needs_layout_passes=False),
    scratch_types=[
        pltpu.VMEM((EPT,), jnp.int32),
        pltpu.VMEM((K,), jnp.int32),
        pltpu.VMEM((K,), jnp.int32),
        pltpu.VMEM((K, D), jnp.float32),
        pltpu.VMEM((K, D), jnp.float32),
        pltpu.VMEM_SHARED((NROWS, D), jnp.float32),
        pltpu.SemaphoreType.DMA,
        pltpu.SemaphoreType.DMA,
        pltpu.SemaphoreType.DMA,
        pltpu.SemaphoreType.DMA,
    ],
)
def _prop_kernel(g_hbm, src_hbm, dst_hbm, out_hbm,
                 sidx, didx0, didx1, rows0, rows1, acc,
                 gsem0, gsem1, dsem0, dsem1):
    cid = lax.axis_index("c")
    sid = lax.axis_index("s")
    tid = cid * NS + sid
    base = pl.multiple_of(tid * EPT, 8)

    zero16 = jnp.zeros((16,), jnp.float32)

    @pl.loop(0, K)
    def _(r):
        for cc in range(D // 16):
            rows0[r, pl.ds(cc * 16, 16)] = zero16

    for j in range(ROWS_PER_TILE // K):
        pltpu.sync_copy(rows0, acc.at[pl.ds(sid * ROWS_PER_TILE + j * K, K)])
    pltpu.sync_copy(src_hbm.at[pl.ds(base, EPT)], sidx)
    pltpu.async_copy(dst_hbm.at[pl.ds(base, K)], didx0, dsem0)
    pltpu.async_copy(g_hbm.at[sidx.at[pl.ds(0, K)]], rows0, gsem0)
    plsc.subcore_barrier()

    @pl.loop(0, NCHUNK // 2)
    def _(i):
        c = i * 2
        o1 = pl.multiple_of((c + 1) * K, 8)
        o2 = pl.multiple_of((c + 2) * K, 8)
        pltpu.async_copy(dst_hbm.at[pl.ds(base + o1, K)], didx1, dsem1)
        pltpu.async_copy(g_hbm.at[sidx.at[pl.ds(o1, K)]], rows1, gsem1)
        pltpu.make_async_copy(g_hbm.at[sidx.at[pl.ds(0, K)]], rows0, gsem0).wait()
        pltpu.make_async_copy(dst_hbm.at[pl.ds(base, K)], didx0, dsem0).wait()
        pltpu.sync_copy(rows0, acc.at[didx0], add=True)
        pltpu.async_copy(dst_hbm.at[pl.ds(base + o2, K)], didx0, dsem0)
        pltpu.async_copy(g_hbm.at[sidx.at[pl.ds(o2, K)]], rows0, gsem0)
        pltpu.make_async_copy(g_hbm.at[sidx.at[pl.ds(0, K)]], rows1, gsem1).wait()
        pltpu.make_async_copy(dst_hbm.at[pl.ds(base, K)], didx1, dsem1).wait()
        pltpu.sync_copy(rows1, acc.at[didx1], add=True)

    pltpu.make_async_copy(g_hbm.at[sidx.at[pl.ds(0, K)]], rows0, gsem0).wait()
    pltpu.make_async_copy(dst_hbm.at[pl.ds(base, K)], didx0, dsem0).wait()
    pltpu.sync_copy(rows0, acc.at[didx0], add=True)

    plsc.subcore_barrier()
    pltpu.sync_copy(acc.at[pl.ds(sid * ROWS_PER_TILE, ROWS_PER_TILE)],
                    out_hbm.at[cid, pl.ds(sid * ROWS_PER_TILE, ROWS_PER_TILE)])


BS = 2000
GR = N // BS


def _mm_scale_body(x_ref, w_ref, d0_ref, d1_ref, o_ref):
    dinv = lax.rsqrt(d0_ref[0] + d1_ref[0] + 1.0)
    h = jnp.dot(x_ref[...], w_ref[...], preferred_element_type=jnp.float32)
    o_ref[...] = dinv * h


def _mm_scale(x, w, degp):
    return pl.pallas_call(
        _mm_scale_body,
        grid=(GR,),
        in_specs=[
            pl.BlockSpec((BS, D), lambda i: (i, 0)),
            pl.BlockSpec((D, D), lambda i: (0, 0)),
            pl.BlockSpec((1, BS, 1), lambda i: (0, i, 0)),
            pl.BlockSpec((1, BS, 1), lambda i: (1, i, 0)),
        ],
        out_specs=pl.BlockSpec((BS, D), lambda i: (i, 0)),
        out_shape=jax.ShapeDtypeStruct((N, D), jnp.float32),
    )(x, w, degp, degp)


def _layer2_body(p0_ref, p1_ref, g_ref, d0_ref, d1_ref, b_ref, w_ref, o_ref):
    dinv = lax.rsqrt(d0_ref[0] + d1_ref[0] + 1.0)
    z = dinv * (p0_ref[0] + p1_ref[0] + g_ref[...]) + b_ref[...]
    z = jnp.maximum(z, 0.0)
    h = jnp.dot(z, w_ref[...], preferred_element_type=jnp.float32)
    o_ref[...] = dinv * h


def _layer2(p, g, degp, b, w):
    return pl.pallas_call(
        _layer2_body,
        grid=(GR,),
        in_specs=[
            pl.BlockSpec((1, BS, D), lambda i: (0, i, 0)),
            pl.BlockSpec((1, BS, D), lambda i: (1, i, 0)),
            pl.BlockSpec((BS, D), lambda i: (i, 0)),
            pl.BlockSpec((1, BS, 1), lambda i: (0, i, 0)),
            pl.BlockSpec((1, BS, 1), lambda i: (1, i, 0)),
            pl.BlockSpec((1, D), lambda i: (0, 0)),
            pl.BlockSpec((D, D), lambda i: (0, 0)),
        ],
        out_specs=pl.BlockSpec((BS, D), lambda i: (i, 0)),
        out_shape=jax.ShapeDtypeStruct((N, D), jnp.float32),
    )(p, p, g, degp, degp, b, w)


def _final_body(p0_ref, p1_ref, g_ref, d0_ref, d1_ref, b_ref, o_ref):
    dinv = lax.rsqrt(d0_ref[0] + d1_ref[0] + 1.0)
    z = dinv * (p0_ref[0] + p1_ref[0] + g_ref[...]) + b_ref[...]
    m = jnp.max(z, axis=1, keepdims=True)
    zc = z - m
    o_ref[...] = zc - jnp.log(jnp.sum(jnp.exp(zc), axis=1, keepdims=True))


def _final(p, g, degp, b):
    return pl.pallas_call(
        _final_body,
        grid=(GR,),
        in_specs=[
            pl.BlockSpec((1, BS, D), lambda i: (0, i, 0)),
            pl.BlockSpec((1, BS, D), lambda i: (1, i, 0)),
            pl.BlockSpec((BS, D), lambda i: (i, 0)),
            pl.BlockSpec((1, BS, 1), lambda i: (0, i, 0)),
            pl.BlockSpec((1, BS, 1), lambda i: (1, i, 0)),
            pl.BlockSpec((1, D), lambda i: (0, 0)),
        ],
        out_specs=pl.BlockSpec((BS, D), lambda i: (i, 0)),
        out_shape=jax.ShapeDtypeStruct((N, D), jnp.float32),
    )(p, p, g, degp, degp, b)


def kernel(x, adj, W1, b1, W2, b2):
    src = adj[0].astype(jnp.int32)
    dst = adj[1].astype(jnp.int32)
    degp = _deg_kernel(dst)
    deg3 = degp.reshape(NC, NPAD, 1)

    b1r = b1.reshape(1, D)
    b2r = b2.reshape(1, D)

    g1 = _mm_scale(x, W1, deg3)
    p = _prop_kernel(g1, src, dst)
    g2 = _layer2(p, g1, deg3, b1r, W2)
    q = _prop_kernel(g2, src, dst)
    return _final(q, g2, deg3, b2r)


# speedup vs baseline: 2.8244x; 1.0079x over previous
import functools

import jax
import jax.numpy as jnp
from jax import lax
from jax.experimental import pallas as pl
from jax.experimental.pallas import tpu as pltpu
from jax.experimental.pallas import tpu_sc as plsc

N = 10000
D = 128
E = 320000
NC = 2
NS = 16
EPT = E // (NC * NS)
K = 80
NCHUNK = EPT // K
NROWS = 10240
ROWS_PER_TILE = NROWS // NS
NPAD = 10240
DEG_STRIPE = NPAD // NS

_mesh = plsc.VectorSubcoreMesh(core_axis_name="c", subcore_axis_name="s")


@functools.partial(
    pl.kernel,
    out_type=jax.ShapeDtypeStruct((NC, NPAD), jnp.float32),
    mesh=_mesh,
    compiler_params=pltpu.CompilerParams(needs_layout_passes=False),
    scratch_types=[
        pltpu.VMEM((NPAD,), jnp.float32),
        pltpu.VMEM((EPT,), jnp.int32),
        pltpu.VMEM((NS, DEG_STRIPE), jnp.float32),
        pltpu.VMEM((DEG_STRIPE,), jnp.float32),
        pltpu.VMEM_SHARED((NS, NPAD), jnp.float32),
        pltpu.SemaphoreType.DMA,
    ],
)
def _deg_kernel(dst_hbm, out_hbm, pdeg, didx, sbuf, obuf, shared_deg, dsem):
    cid = lax.axis_index("c")
    sid = lax.axis_index("s")
    tid = cid * NS + sid

    pltpu.async_copy(dst_hbm.at[pl.ds(tid * EPT, EPT)], didx, dsem)
    zero16 = jnp.zeros((16,), jnp.float32)

    @pl.loop(0, NPAD // 16, unroll=8)
    def _(r):
        pdeg[pl.ds(r * 16, 16)] = zero16

    pltpu.make_async_copy(dst_hbm.at[pl.ds(tid * EPT, EPT)], didx, dsem).wait()
    ones16 = jnp.ones((16,), jnp.float32)

    @pl.loop(0, EPT // 16, unroll=8)
    def _(j):
        idx16 = didx[pl.ds(j * 16, 16)]
        plsc.addupdate_scatter(pdeg, [idx16], ones16)

    pltpu.sync_copy(pdeg, shared_deg.at[sid])
    plsc.subcore_barrier()
    pltpu.sync_copy(shared_deg.at[:, pl.ds(sid * DEG_STRIPE, DEG_STRIPE)], sbuf)

    @pl.loop(0, DEG_STRIPE // 16, unroll=4)
    def _(c):
        acc16 = sbuf[0, pl.ds(c * 16, 16)]
        for w in range(1, NS):
            acc16 += sbuf[w, pl.ds(c * 16, 16)]
        obuf[pl.ds(c * 16, 16)] = acc16

    pltpu.sync_copy(obuf, out_hbm.at[cid, pl.ds(sid * DEG_STRIPE, DEG_STRIPE)])


@functools.partial(
    pl.kernel,
    out_type=jax.ShapeDtypeStruct((NC, NROWS, D), jnp.float32),
    mesh=_mesh,
    compiler_params=pltpu.CompilerParams(needs_layout_passes=False),
    scratch_types=[
        pltpu.VMEM((EPT,), jnp.int32),
        pltpu.VMEM((K,), jnp.int32),
        pltpu.VMEM((K,), jnp.int32),
        pltpu.VMEM((K, D), jnp.float32),
        pltpu.VMEM((K, D), jnp.float32),
        pltpu.VMEM_SHARED((NROWS, D), jnp.float32),
        pltpu.SemaphoreType.DMA,
        pltpu.SemaphoreType.DMA,
        pltpu.SemaphoreType.DMA,
        pltpu.SemaphoreType.DMA,
    ],
)
def _prop_kernel(g_hbm, src_hbm, dst_hbm, out_hbm,
                 sidx, didx0, didx1, rows0, rows1, acc,
                 gsem0, gsem1, dsem0, dsem1):
    cid = lax.axis_index("c")
    sid = lax.axis_index("s")
    tid = cid * NS + sid
    base = pl.multiple_of(tid * EPT, 8)

    zero16 = jnp.zeros((16,), jnp.float32)

    @pl.loop(0, K, unroll=4)
    def _(r):
        for cc in range(D // 16):
            rows0[r, pl.ds(cc * 16, 16)] = zero16

    for j in range(ROWS_PER_TILE // K):
        pltpu.sync_copy(rows0, acc.at[pl.ds(sid * ROWS_PER_TILE + j * K, K)])
    pltpu.sync_copy(src_hbm.at[pl.ds(base, EPT)], sidx)
    pltpu.async_copy(dst_hbm.at[pl.ds(base, K)], didx0, dsem0)
    pltpu.async_copy(g_hbm.at[sidx.at[pl.ds(0, K)]], rows0, gsem0)
    plsc.subcore_barrier()

    @pl.loop(0, NCHUNK // 2)
    def _(i):
        c = i * 2
        o1 = pl.multiple_of((c + 1) * K, 8)
        o2 = pl.multiple_of((c + 2) * K, 8)
        pltpu.async_copy(dst_hbm.at[pl.ds(base + o1, K)], didx1, dsem1)
        pltpu.async_copy(g_hbm.at[sidx.at[pl.ds(o1, K)]], rows1, gsem1)
        pltpu.make_async_copy(g_hbm.at[sidx.at[pl.ds(0, K)]], rows0, gsem0).wait()
        pltpu.make_async_copy(dst_hbm.at[pl.ds(base, K)], didx0, dsem0).wait()
        pltpu.sync_copy(rows0, acc.at[didx0], add=True)
        pltpu.async_copy(dst_hbm.at[pl.ds(base + o2, K)], didx0, dsem0)
        pltpu.async_copy(g_hbm.at[sidx.at[pl.ds(o2, K)]], rows0, gsem0)
        pltpu.make_async_copy(g_hbm.at[sidx.at[pl.ds(0, K)]], rows1, gsem1).wait()
        pltpu.make_async_copy(dst_hbm.at[pl.ds(base, K)], didx1, dsem1).wait()
        pltpu.sync_copy(rows1, acc.at[didx1], add=True)

    pltpu.make_async_copy(g_hbm.at[sidx.at[pl.ds(0, K)]], rows0, gsem0).wait()
    pltpu.make_async_copy(dst_hbm.at[pl.ds(base, K)], didx0, dsem0).wait()
    pltpu.sync_copy(rows0, acc.at[didx0], add=True)

    plsc.subcore_barrier()
    pltpu.sync_copy(acc.at[pl.ds(sid * ROWS_PER_TILE, ROWS_PER_TILE)],
                    out_hbm.at[cid, pl.ds(sid * ROWS_PER_TILE, ROWS_PER_TILE)])


BS = 2000
GR = N // BS


def _mm_scale_body(x_ref, w_ref, d0_ref, d1_ref, o_ref):
    dinv = lax.rsqrt(d0_ref[0] + d1_ref[0] + 1.0)
    h = jnp.dot(x_ref[...], w_ref[...], preferred_element_type=jnp.float32)
    o_ref[...] = dinv * h


def _mm_scale(x, w, degp):
    return pl.pallas_call(
        _mm_scale_body,
        grid=(GR,),
        in_specs=[
            pl.BlockSpec((BS, D), lambda i: (i, 0)),
            pl.BlockSpec((D, D), lambda i: (0, 0)),
            pl.BlockSpec((1, BS, 1), lambda i: (0, i, 0)),
            pl.BlockSpec((1, BS, 1), lambda i: (1, i, 0)),
        ],
        out_specs=pl.BlockSpec((BS, D), lambda i: (i, 0)),
        out_shape=jax.ShapeDtypeStruct((N, D), jnp.float32),
    )(x, w, degp, degp)


def _layer2_body(p0_ref, p1_ref, g_ref, d0_ref, d1_ref, b_ref, w_ref, o_ref):
    dinv = lax.rsqrt(d0_ref[0] + d1_ref[0] + 1.0)
    z = dinv * (p0_ref[0] + p1_ref[0] + g_ref[...]) + b_ref[...]
    z = jnp.maximum(z, 0.0)
    h = jnp.dot(z, w_ref[...], preferred_element_type=jnp.float32)
    o_ref[...] = dinv * h


def _layer2(p, g, degp, b, w):
    return pl.pallas_call(
        _layer2_body,
        grid=(GR,),
        in_specs=[
            pl.BlockSpec((1, BS, D), lambda i: (0, i, 0)),
            pl.BlockSpec((1, BS, D), lambda i: (1, i, 0)),
            pl.BlockSpec((BS, D), lambda i: (i, 0)),
            pl.BlockSpec((1, BS, 1), lambda i: (0, i, 0)),
            pl.BlockSpec((1, BS, 1), lambda i: (1, i, 0)),
            pl.BlockSpec((1, D), lambda i: (0, 0)),
            pl.BlockSpec((D, D), lambda i: (0, 0)),
        ],
        out_specs=pl.BlockSpec((BS, D), lambda i: (i, 0)),
        out_shape=jax.ShapeDtypeStruct((N, D), jnp.float32),
    )(p, p, g, degp, degp, b, w)


def _final_body(p0_ref, p1_ref, g_ref, d0_ref, d1_ref, b_ref, o_ref):
    dinv = lax.rsqrt(d0_ref[0] + d1_ref[0] + 1.0)
    z = dinv * (p0_ref[0] + p1_ref[0] + g_ref[...]) + b_ref[...]
    m = jnp.max(z, axis=1, keepdims=True)
    zc = z - m
    o_ref[...] = zc - jnp.log(jnp.sum(jnp.exp(zc), axis=1, keepdims=True))


def _final(p, g, degp, b):
    return pl.pallas_call(
        _final_body,
        grid=(GR,),
        in_specs=[
            pl.BlockSpec((1, BS, D), lambda i: (0, i, 0)),
            pl.BlockSpec((1, BS, D), lambda i: (1, i, 0)),
            pl.BlockSpec((BS, D), lambda i: (i, 0)),
            pl.BlockSpec((1, BS, 1), lambda i: (0, i, 0)),
            pl.BlockSpec((1, BS, 1), lambda i: (1, i, 0)),
            pl.BlockSpec((1, D), lambda i: (0, 0)),
        ],
        out_specs=pl.BlockSpec((BS, D), lambda i: (i, 0)),
        out_shape=jax.ShapeDtypeStruct((N, D), jnp.float32),
    )(p, p, g, degp, degp, b)


def kernel(x, adj, W1, b1, W2, b2):
    src = adj[0].astype(jnp.int32)
    dst = adj[1].astype(jnp.int32)
    degp = _deg_kernel(dst)
    deg3 = degp.reshape(NC, NPAD, 1)

    b1r = b1.reshape(1, D)
    b2r = b2.reshape(1, D)

    g1 = _mm_scale(x, W1, deg3)
    p = _prop_kernel(g1, src, dst)
    g2 = _layer2(p, g1, deg3, b1r, W2)
    q = _prop_kernel(g2, src, dst)
    return _final(q, g2, deg3, b2r)


# speedup vs baseline: 2.8359x; 1.0041x over previous
import functools

import jax
import jax.numpy as jnp
from jax import lax
from jax.experimental import pallas as pl
from jax.experimental.pallas import tpu as pltpu
from jax.experimental.pallas import tpu_sc as plsc

N = 10000
D = 128
E = 320000
NC = 2
NS = 16
EPT = E // (NC * NS)
K = 80
NCHUNK = EPT // K
NROWS = 10240
ROWS_PER_TILE = NROWS // NS
NPAD = 10240
DEG_STRIPE = NPAD // NS

_mesh = plsc.VectorSubcoreMesh(core_axis_name="c", subcore_axis_name="s")


@functools.partial(
    pl.kernel,
    out_type=jax.ShapeDtypeStruct((NC, NPAD), jnp.float32),
    mesh=_mesh,
    compiler_params=pltpu.CompilerParams(needs_layout_passes=False),
    scratch_types=[
        pltpu.VMEM((NPAD,), jnp.float32),
        pltpu.VMEM((EPT,), jnp.int32),
        pltpu.VMEM((NS, DEG_STRIPE), jnp.float32),
        pltpu.VMEM((DEG_STRIPE,), jnp.float32),
        pltpu.VMEM_SHARED((NS, NPAD), jnp.float32),
        pltpu.SemaphoreType.DMA,
    ],
)
def _deg_kernel(dst_hbm, out_hbm, pdeg, didx, sbuf, obuf, shared_deg, dsem):
    cid = lax.axis_index("c")
    sid = lax.axis_index("s")
    tid = cid * NS + sid

    pltpu.async_copy(dst_hbm.at[pl.ds(tid * EPT, EPT)], didx, dsem)
    zero16 = jnp.zeros((16,), jnp.float32)

    @pl.loop(0, NPAD // 16, unroll=8)
    def _(r):
        pdeg[pl.ds(r * 16, 16)] = zero16

    pltpu.make_async_copy(dst_hbm.at[pl.ds(tid * EPT, EPT)], didx, dsem).wait()
    ones16 = jnp.ones((16,), jnp.float32)

    @pl.loop(0, EPT // 16, unroll=8)
    def _(j):
        idx16 = didx[pl.ds(j * 16, 16)]
        plsc.addupdate_scatter(pdeg, [idx16], ones16)

    pltpu.sync_copy(pdeg, shared_deg.at[sid])
    plsc.subcore_barrier()
    pltpu.sync_copy(shared_deg.at[:, pl.ds(sid * DEG_STRIPE, DEG_STRIPE)], sbuf)

    @pl.loop(0, DEG_STRIPE // 16, unroll=4)
    def _(c):
        acc16 = sbuf[0, pl.ds(c * 16, 16)]
        for w in range(1, NS):
            acc16 += sbuf[w, pl.ds(c * 16, 16)]
        obuf[pl.ds(c * 16, 16)] = acc16

    pltpu.sync_copy(obuf, out_hbm.at[cid, pl.ds(sid * DEG_STRIPE, DEG_STRIPE)])


@functools.partial(
    pl.kernel,
    out_type=jax.ShapeDtypeStruct((NC, NROWS, D), jnp.float32),
    mesh=_mesh,
    compiler_params=pltpu.CompilerParams(needs_layout_passes=False),
    scratch_types=[
        pltpu.VMEM((EPT,), jnp.int32),
        pltpu.VMEM((K,), jnp.int32),
        pltpu.VMEM((K,), jnp.int32),
        pltpu.VMEM((K, D), jnp.float32),
        pltpu.VMEM((K, D), jnp.float32),
        pltpu.VMEM_SHARED((NROWS, D), jnp.float32),
        pltpu.SemaphoreType.DMA,
        pltpu.SemaphoreType.DMA,
        pltpu.SemaphoreType.DMA,
        pltpu.SemaphoreType.DMA,
    ],
)
def _prop_kernel(g_hbm, src_hbm, dst_hbm, out_hbm,
                 sidx, didx0, didx1, rows0, rows1, acc,
                 gsem0, gsem1, dsem0, dsem1):
    cid = lax.axis_index("c")
    sid = lax.axis_index("s")
    tid = cid * NS + sid
    base = pl.multiple_of(tid * EPT, 8)

    zero16 = jnp.zeros((16,), jnp.float32)

    @pl.loop(0, K, unroll=4)
    def _(r):
        for cc in range(D // 16):
            rows0[r, pl.ds(cc * 16, 16)] = zero16

    for j in range(ROWS_PER_TILE // K):
        pltpu.sync_copy(rows0, acc.at[pl.ds(sid * ROWS_PER_TILE + j * K, K)])
    pltpu.sync_copy(src_hbm.at[pl.ds(base, EPT)], sidx)
    pltpu.async_copy(dst_hbm.at[pl.ds(base, K)], didx0, dsem0)
    pltpu.async_copy(g_hbm.at[sidx.at[pl.ds(0, K)]], rows0, gsem0)
    plsc.subcore_barrier()

    @pl.loop(0, NCHUNK // 2)
    def _(i):
        c = i * 2
        o1 = pl.multiple_of((c + 1) * K, 8)
        o2 = pl.multiple_of((c + 2) * K, 8)
        pltpu.async_copy(dst_hbm.at[pl.ds(base + o1, K)], didx1, dsem1)
        pltpu.async_copy(g_hbm.at[sidx.at[pl.ds(o1, K)]], rows1, gsem1)
        pltpu.make_async_copy(g_hbm.at[sidx.at[pl.ds(0, K)]], rows0, gsem0).wait()
        pltpu.make_async_copy(dst_hbm.at[pl.ds(base, K)], didx0, dsem0).wait()
        pltpu.sync_copy(rows0, acc.at[didx0], add=True)
        pltpu.async_copy(dst_hbm.at[pl.ds(base + o2, K)], didx0, dsem0)
        pltpu.async_copy(g_hbm.at[sidx.at[pl.ds(o2, K)]], rows0, gsem0)
        pltpu.make_async_copy(g_hbm.at[sidx.at[pl.ds(0, K)]], rows1, gsem1).wait()
        pltpu.make_async_copy(dst_hbm.at[pl.ds(base, K)], didx1, dsem1).wait()
        pltpu.sync_copy(rows1, acc.at[didx1], add=True)

    pltpu.make_async_copy(g_hbm.at[sidx.at[pl.ds(0, K)]], rows0, gsem0).wait()
    pltpu.make_async_copy(dst_hbm.at[pl.ds(base, K)], didx0, dsem0).wait()
    pltpu.sync_copy(rows0, acc.at[didx0], add=True)

    plsc.subcore_barrier()
    pltpu.sync_copy(acc.at[pl.ds(sid * ROWS_PER_TILE, ROWS_PER_TILE)],
                    out_hbm.at[cid, pl.ds(sid * ROWS_PER_TILE, ROWS_PER_TILE)])


BS = 5000
GR = N // BS


def _mm_scale_body(x_ref, w_ref, d0_ref, d1_ref, o_ref):
    dinv = lax.rsqrt(d0_ref[0] + d1_ref[0] + 1.0)
    h = jnp.dot(x_ref[...], w_ref[...], preferred_element_type=jnp.float32)
    o_ref[...] = dinv * h


def _mm_scale(x, w, degp):
    return pl.pallas_call(
        _mm_scale_body,
        grid=(GR,),
        in_specs=[
            pl.BlockSpec((BS, D), lambda i: (i, 0)),
            pl.BlockSpec((D, D), lambda i: (0, 0)),
            pl.BlockSpec((1, BS, 1), lambda i: (0, i, 0)),
            pl.BlockSpec((1, BS, 1), lambda i: (1, i, 0)),
        ],
        out_specs=pl.BlockSpec((BS, D), lambda i: (i, 0)),
        out_shape=jax.ShapeDtypeStruct((N, D), jnp.float32),
    )(x, w, degp, degp)


def _layer2_body(p0_ref, p1_ref, g_ref, d0_ref, d1_ref, b_ref, w_ref, o_ref):
    dinv = lax.rsqrt(d0_ref[0] + d1_ref[0] + 1.0)
    z = dinv * (p0_ref[0] + p1_ref[0] + g_ref[...]) + b_ref[...]
    z = jnp.maximum(z, 0.0)
    h = jnp.dot(z, w_ref[...], preferred_element_type=jnp.float32)
    o_ref[...] = dinv * h


def _layer2(p, g, degp, b, w):
    return pl.pallas_call(
        _layer2_body,
        grid=(GR,),
        in_specs=[
            pl.BlockSpec((1, BS, D), lambda i: (0, i, 0)),
            pl.BlockSpec((1, BS, D), lambda i: (1, i, 0)),
            pl.BlockSpec((BS, D), lambda i: (i, 0)),
            pl.BlockSpec((1, BS, 1), lambda i: (0, i, 0)),
            pl.BlockSpec((1, BS, 1), lambda i: (1, i, 0)),
            pl.BlockSpec((1, D), lambda i: (0, 0)),
            pl.BlockSpec((D, D), lambda i: (0, 0)),
        ],
        out_specs=pl.BlockSpec((BS, D), lambda i: (i, 0)),
        out_shape=jax.ShapeDtypeStruct((N, D), jnp.float32),
    )(p, p, g, degp, degp, b, w)


def _final_body(p0_ref, p1_ref, g_ref, d0_ref, d1_ref, b_ref, o_ref):
    dinv = lax.rsqrt(d0_ref[0] + d1_ref[0] + 1.0)
    z = dinv * (p0_ref[0] + p1_ref[0] + g_ref[...]) + b_ref[...]
    m = jnp.max(z, axis=1, keepdims=True)
    zc = z - m
    o_ref[...] = zc - jnp.log(jnp.sum(jnp.exp(zc), axis=1, keepdims=True))


def _final(p, g, degp, b):
    return pl.pallas_call(
        _final_body,
        grid=(GR,),
        in_specs=[
            pl.BlockSpec((1, BS, D), lambda i: (0, i, 0)),
            pl.BlockSpec((1, BS, D), lambda i: (1, i, 0)),
            pl.BlockSpec((BS, D), lambda i: (i, 0)),
            pl.BlockSpec((1, BS, 1), lambda i: (0, i, 0)),
            pl.BlockSpec((1, BS, 1), lambda i: (1, i, 0)),
            pl.BlockSpec((1, D), lambda i: (0, 0)),
        ],
        out_specs=pl.BlockSpec((BS, D), lambda i: (i, 0)),
        out_shape=jax.ShapeDtypeStruct((N, D), jnp.float32),
    )(p, p, g, degp, degp, b)


def kernel(x, adj, W1, b1, W2, b2):
    src = adj[0].astype(jnp.int32)
    dst = adj[1].astype(jnp.int32)
    degp = _deg_kernel(dst)
    deg3 = degp.reshape(NC, NPAD, 1)

    b1r = b1.reshape(1, D)
    b2r = b2.reshape(1, D)

    g1 = _mm_scale(x, W1, deg3)
    p = _prop_kernel(g1, src, dst)
    g2 = _layer2(p, g1, deg3, b1r, W2)
    q = _prop_kernel(g2, src, dst)
    return _final(q, g2, deg3, b2r)
